# baseline (speedup 1.0000x reference)
import functools

import jax
import jax.numpy as jnp
from jax import lax
from jax.experimental import pallas as pl
from jax.experimental.pallas import tpu as pltpu
from jax.experimental.pallas import tpu_sc as plsc

VOCAB = 1000000
D_EMB = 64
H = 256
OUT = 128
B = 4096
LQ = 20
LD = 200
EPS = 1e-5

NC = 2
NS = 16
NW = NC * NS
LANES = 16
DV = D_EMB // LANES


D_PAD = 128


def _make_pool_kernel(L: int, nseg: int, cb: int, unroll: int):
  seg = L // nseg
  scale = 1.0 / L
  bags_pw = B // NW
  nch = bags_pw // cb
  assert nch % 2 == 0 and seg % unroll == 0 and seg <= 128
  assert nseg == 1 or seg % 8 == 0
  k = cb * nseg

  mesh = plsc.VectorSubcoreMesh(core_axis_name="c", subcore_axis_name="s")

  @functools.partial(
      pl.kernel,
      mesh=mesh,
      compiler_params=pltpu.CompilerParams(use_tc_tiling_on_sc=False),
      out_type=jax.ShapeDtypeStruct((B, D_EMB), jnp.float32),
      scratch_types=[
          pltpu.VMEM((2, cb, L), jnp.int32),
          pltpu.VMEM((2, k, seg, D_EMB), jnp.float32),
          pltpu.VMEM((cb, D_EMB), jnp.float32),
          pltpu.SemaphoreType.DMA,
          pltpu.SemaphoreType.DMA,
      ],
  )
  def pool(tok_hbm, table_hbm, out_hbm, idx_v, rows_v, out_s, sem0, sem1):
    wid = lax.axis_index("s") * NC + lax.axis_index("c")
    bag_base = wid * bags_pw
    sems = (sem0, sem1)

    def copies(ch, buf):
      bag0 = bag_base + ch * cb
      idx_view = (lambda b, h: idx_v.at[buf, b]) if nseg == 1 else (
          lambda b, h: idx_v.at[buf, b, pl.ds(h * seg, seg)])
      return [
          pltpu.make_async_copy(
              table_hbm.at[idx_view(b, h)],
              rows_v.at[buf, b * nseg + h],
              sems[buf])
          for b in range(cb) for h in range(nseg)
      ]

    def issue(ch, buf):
      pltpu.sync_copy(tok_hbm.at[pl.ds(bag_base + ch * cb, cb)],
                      idx_v.at[buf])
      for c in copies(ch, buf):
        c.start()

    def consume(ch, buf):
      for c in copies(ch, buf):
        c.wait()
      for b in range(cb):
        def acc_body(r, accs):
          new = list(accs)
          for u in range(unroll):
            for j in range(nseg):
              for c in range(DV):
                new[c] = new[c] + rows_v[buf, b * nseg + j, r * unroll + u,
                                         pl.ds(c * LANES, LANES)]
          return tuple(new)
        accs = lax.fori_loop(
            0, seg // unroll, acc_body,
            tuple(jnp.zeros((LANES,), jnp.float32) for _ in range(DV)))
        for c in range(DV):
          out_s[b, pl.ds(c * LANES, LANES)] = accs[c] * scale
      pltpu.sync_copy(out_s, out_hbm.at[pl.ds(bag_base + ch * cb, cb)])

    issue(0, 0)

    def pair_body(i, carry):
      ch0 = 2 * i
      issue(ch0 + 1, 1)
      consume(ch0, 0)

      @pl.when(ch0 + 2 < nch)
      def _():
        issue(ch0 + 2, 0)

      consume(ch0 + 1, 1)
      return carry

    lax.fori_loop(0, nch // 2, pair_body, 0)

  return pool


_pool_q = _make_pool_kernel(L=LQ, nseg=1, cb=16, unroll=4)
_pool_d = _make_pool_kernel(L=LD, nseg=5, cb=4, unroll=2)


TB = 8192
_N_TB = -(-VOCAB // TB)


def _repack_body(tt_ref, out_ref):
  out_ref[:, 0:D_EMB] = lax.transpose(tt_ref[:, :], (1, 0))


def _repack(tt):
  padded = pl.pallas_call(
      _repack_body,
      grid=(_N_TB,),
      in_specs=[pl.BlockSpec((D_EMB, TB), lambda i: (0, i))],
      out_specs=pl.BlockSpec((TB, D_PAD), lambda i: (i, 0)),
      out_shape=jax.ShapeDtypeStruct((VOCAB, D_PAD), jnp.float32),
  )(tt)
  return padded.reshape(2 * VOCAB, D_EMB)


BLK = 256


def _mlp_body(x, W1, b1, g1, be1, Wo, bo, o):
  h = jnp.dot(x[:, :], W1[:, :], preferred_element_type=jnp.float32)
  h = jnp.maximum(h + b1[:, :], 0.0)
  mu = jnp.mean(h, axis=-1, keepdims=True)
  var = jnp.mean((h - mu) ** 2, axis=-1, keepdims=True)
  hn = (h - mu) * lax.rsqrt(var + EPS) * g1[:, :] + be1[:, :]
  o[:, :] = jnp.dot(hn, Wo[:, :], preferred_element_type=jnp.float32) + bo[:, :]


def _mlp(x, W1, b1, g1, be1, Wo, bo):
  full = lambda shape: pl.BlockSpec(shape, lambda i: (0, 0))
  return pl.pallas_call(
      _mlp_body,
      grid=(B // BLK,),
      in_specs=[
          pl.BlockSpec((BLK, D_EMB), lambda i: (i, 0)),
          full((D_EMB, H)), full((1, H)), full((1, H)), full((1, H)),
          full((H, OUT)), full((1, OUT)),
      ],
      out_specs=pl.BlockSpec((BLK, OUT), lambda i: (i, 0)),
      out_shape=jax.ShapeDtypeStruct((B, OUT), jnp.float32),
  )(x, W1, b1, g1, be1, Wo, bo)


def kernel(query_tokens, doc_tokens, emb_table,
           W1_q, b1_q, g1_q, be1_q, Wo_q, bo_q,
           W1_d, b1_d, g1_d, be1_d, Wo_d, bo_d):
  qt = query_tokens.astype(jnp.int32) * 2
  dt = doc_tokens.astype(jnp.int32) * 2
  tbl = _repack(emb_table.T)
  pooled_d = _pool_d(dt, tbl)
  pooled_q = _pool_q(qt, tbl)
  row = lambda v: v.reshape(1, -1)
  d = _mlp(pooled_d, W1_d, row(b1_d), row(g1_d), row(be1_d), Wo_d, row(bo_d))
  q = _mlp(pooled_q, W1_q, row(b1_q), row(g1_q), row(be1_q), Wo_q, row(bo_q))
  return (q, d)


# speedup vs baseline: 1.0435x; 1.0435x over previous
import functools

import jax
import jax.numpy as jnp
from jax import lax
from jax.experimental import pallas as pl
from jax.experimental.pallas import tpu as pltpu
from jax.experimental.pallas import tpu_sc as plsc

VOCAB = 1000000
D_EMB = 64
H = 256
OUT = 128
B = 4096
LQ = 20
LD = 200
EPS = 1e-5

NC = 2
NS = 16
NW = NC * NS
LANES = 16
DV = D_EMB // LANES


D_PAD = 128


def _make_pool_kernel(L: int, nseg: int, cb: int, unroll: int):
  seg = L // nseg
  scale = 1.0 / L
  bags_pw = B // NW
  nch = bags_pw // cb
  assert nch % 2 == 0 and seg % unroll == 0 and seg <= 128
  assert nseg == 1 or seg % 8 == 0
  k = cb * nseg

  mesh = plsc.VectorSubcoreMesh(core_axis_name="c", subcore_axis_name="s")

  @functools.partial(
      pl.kernel,
      mesh=mesh,
      compiler_params=pltpu.CompilerParams(use_tc_tiling_on_sc=False),
      out_type=jax.ShapeDtypeStruct((B, D_EMB), jnp.float32),
      scratch_types=[
          pltpu.VMEM((2, cb, L), jnp.int32),
          pltpu.VMEM((2, k, seg, D_EMB), jnp.float32),
          pltpu.VMEM((cb, D_EMB), jnp.float32),
          pltpu.SemaphoreType.DMA,
          pltpu.SemaphoreType.DMA,
      ],
  )
  def pool(tok_hbm, table_hbm, out_hbm, idx_v, rows_v, out_s, sem0, sem1):
    wid = lax.axis_index("s") * NC + lax.axis_index("c")
    bag_base = wid * bags_pw
    sems = (sem0, sem1)

    def copies(ch, buf):
      bag0 = bag_base + ch * cb
      idx_view = (lambda b, h: idx_v.at[buf, b]) if nseg == 1 else (
          lambda b, h: idx_v.at[buf, b, pl.ds(h * seg, seg)])
      return [
          pltpu.make_async_copy(
              table_hbm.at[idx_view(b, h)],
              rows_v.at[buf, b * nseg + h],
              sems[buf])
          for b in range(cb) for h in range(nseg)
      ]

    def issue(ch, buf):
      pltpu.sync_copy(tok_hbm.at[pl.ds(bag_base + ch * cb, cb)],
                      idx_v.at[buf])
      for c in copies(ch, buf):
        c.start()

    def consume(ch, buf):
      for c in copies(ch, buf):
        c.wait()
      for b in range(cb):
        def acc_body(r, accs):
          new = list(accs)
          for u in range(unroll):
            for j in range(nseg):
              for c in range(DV):
                new[c] = new[c] + rows_v[buf, b * nseg + j, r * unroll + u,
                                         pl.ds(c * LANES, LANES)]
          return tuple(new)
        accs = lax.fori_loop(
            0, seg // unroll, acc_body,
            tuple(jnp.zeros((LANES,), jnp.float32) for _ in range(DV)))
        for c in range(DV):
          out_s[b, pl.ds(c * LANES, LANES)] = accs[c] * scale
      pltpu.sync_copy(out_s, out_hbm.at[pl.ds(bag_base + ch * cb, cb)])

    issue(0, 0)

    def pair_body(i, carry):
      ch0 = 2 * i
      issue(ch0 + 1, 1)
      consume(ch0, 0)

      @pl.when(ch0 + 2 < nch)
      def _():
        issue(ch0 + 2, 0)

      consume(ch0 + 1, 1)
      return carry

    lax.fori_loop(0, nch // 2, pair_body, 0)

  return pool


_pool_q = _make_pool_kernel(L=LQ, nseg=1, cb=16, unroll=4)
_pool_d = _make_pool_kernel(L=LD, nseg=5, cb=4, unroll=2)


TB = 16384
_N_TB = -(-VOCAB // TB)


def _repack_body(tt_ref, out_ref):
  out_ref[:, 0:D_EMB] = lax.transpose(tt_ref[:, :], (1, 0))


def _repack(tt):
  padded = pl.pallas_call(
      _repack_body,
      grid=(_N_TB,),
      in_specs=[pl.BlockSpec((D_EMB, TB), lambda i: (0, i))],
      out_specs=pl.BlockSpec((TB, D_PAD), lambda i: (i, 0)),
      out_shape=jax.ShapeDtypeStruct((VOCAB, D_PAD), jnp.float32),
  )(tt)
  return padded.reshape(2 * VOCAB, D_EMB)


BLK = 256


def _mlp_body(x, W1, b1, g1, be1, Wo, bo, o):
  h = jnp.dot(x[:, :], W1[:, :], preferred_element_type=jnp.float32)
  h = jnp.maximum(h + b1[:, :], 0.0)
  mu = jnp.mean(h, axis=-1, keepdims=True)
  var = jnp.mean((h - mu) ** 2, axis=-1, keepdims=True)
  hn = (h - mu) * lax.rsqrt(var + EPS) * g1[:, :] + be1[:, :]
  o[:, :] = jnp.dot(hn, Wo[:, :], preferred_element_type=jnp.float32) + bo[:, :]


def _mlp(x, W1, b1, g1, be1, Wo, bo):
  full = lambda shape: pl.BlockSpec(shape, lambda i: (0, 0))
  return pl.pallas_call(
      _mlp_body,
      grid=(B // BLK,),
      in_specs=[
          pl.BlockSpec((BLK, D_EMB), lambda i: (i, 0)),
          full((D_EMB, H)), full((1, H)), full((1, H)), full((1, H)),
          full((H, OUT)), full((1, OUT)),
      ],
      out_specs=pl.BlockSpec((BLK, OUT), lambda i: (i, 0)),
      out_shape=jax.ShapeDtypeStruct((B, OUT), jnp.float32),
  )(x, W1, b1, g1, be1, Wo, bo)


def kernel(query_tokens, doc_tokens, emb_table,
           W1_q, b1_q, g1_q, be1_q, Wo_q, bo_q,
           W1_d, b1_d, g1_d, be1_d, Wo_d, bo_d):
  qt = query_tokens.astype(jnp.int32) * 2
  dt = doc_tokens.astype(jnp.int32) * 2
  tbl = _repack(emb_table.T)
  pooled_d = _pool_d(dt, tbl)
  pooled_q = _pool_q(qt, tbl)
  row = lambda v: v.reshape(1, -1)
  d = _mlp(pooled_d, W1_d, row(b1_d), row(g1_d), row(be1_d), Wo_d, row(bo_d))
  q = _mlp(pooled_q, W1_q, row(b1_q), row(g1_q), row(be1_q), Wo_q, row(bo_q))
  return (q, d)


# speedup vs baseline: 1.0588x; 1.0146x over previous
import functools

import jax
import jax.numpy as jnp
from jax import lax
from jax.experimental import pallas as pl
from jax.experimental.pallas import tpu as pltpu
from jax.experimental.pallas import tpu_sc as plsc

VOCAB = 1000000
D_EMB = 64
H = 256
OUT = 128
B = 4096
LQ = 20
LD = 200
EPS = 1e-5

NC = 2
NS = 16
NW = NC * NS
LANES = 16
DV = D_EMB // LANES


D_PAD = 128


def _make_pool_kernel(L: int, nseg: int, cb: int, unroll: int):
  seg = L // nseg
  scale = 1.0 / L
  bags_pw = B // NW
  nch = bags_pw // cb
  assert nch % 2 == 0 and seg % unroll == 0 and seg <= 128
  assert nseg == 1 or seg % 8 == 0
  k = cb * nseg

  mesh = plsc.VectorSubcoreMesh(core_axis_name="c", subcore_axis_name="s")

  @functools.partial(
      pl.kernel,
      mesh=mesh,
      compiler_params=pltpu.CompilerParams(use_tc_tiling_on_sc=False),
      out_type=jax.ShapeDtypeStruct((B, D_EMB), jnp.float32),
      scratch_types=[
          pltpu.VMEM((2, cb, L), jnp.int32),
          pltpu.VMEM((2, k, seg, D_EMB), jnp.float32),
          pltpu.VMEM((cb, D_EMB), jnp.float32),
          pltpu.SemaphoreType.DMA,
          pltpu.SemaphoreType.DMA,
      ],
  )
  def pool(tok_hbm, table_hbm, out_hbm, idx_v, rows_v, out_s, sem0, sem1):
    wid = lax.axis_index("s") * NC + lax.axis_index("c")
    bag_base = wid * bags_pw
    sems = (sem0, sem1)

    def copies(ch, buf):
      bag0 = bag_base + ch * cb
      idx_view = (lambda b, h: idx_v.at[buf, b]) if nseg == 1 else (
          lambda b, h: idx_v.at[buf, b, pl.ds(h * seg, seg)])
      return [
          pltpu.make_async_copy(
              table_hbm.at[idx_view(b, h)],
              rows_v.at[buf, b * nseg + h],
              sems[buf])
          for b in range(cb) for h in range(nseg)
      ]

    def issue(ch, buf):
      pltpu.sync_copy(tok_hbm.at[pl.ds(bag_base + ch * cb, cb)],
                      idx_v.at[buf])
      for c in copies(ch, buf):
        c.start()

    def consume(ch, buf):
      for c in copies(ch, buf):
        c.wait()
      for b in range(cb):
        def acc_body(r, accs):
          new = list(accs)
          for u in range(unroll):
            for j in range(nseg):
              for c in range(DV):
                new[c] = new[c] + rows_v[buf, b * nseg + j, r * unroll + u,
                                         pl.ds(c * LANES, LANES)]
          return tuple(new)
        accs = lax.fori_loop(
            0, seg // unroll, acc_body,
            tuple(jnp.zeros((LANES,), jnp.float32) for _ in range(DV)))
        for c in range(DV):
          out_s[b, pl.ds(c * LANES, LANES)] = accs[c] * scale
      pltpu.sync_copy(out_s, out_hbm.at[pl.ds(bag_base + ch * cb, cb)])

    issue(0, 0)

    def pair_body(i, carry):
      ch0 = 2 * i
      issue(ch0 + 1, 1)
      consume(ch0, 0)

      @pl.when(ch0 + 2 < nch)
      def _():
        issue(ch0 + 2, 0)

      consume(ch0 + 1, 1)
      return carry

    lax.fori_loop(0, nch // 2, pair_body, 0)

  return pool


_pool_q = _make_pool_kernel(L=LQ, nseg=1, cb=16, unroll=4)
_pool_d = _make_pool_kernel(L=LD, nseg=5, cb=4, unroll=2)


TB = 32768
_N_TB = -(-VOCAB // TB)


def _repack_body(tt_ref, out_ref):
  out_ref[:, 0:D_EMB] = lax.transpose(tt_ref[:, :], (1, 0))


def _repack(tt):
  padded = pl.pallas_call(
      _repack_body,
      grid=(_N_TB,),
      in_specs=[pl.BlockSpec((D_EMB, TB), lambda i: (0, i))],
      out_specs=pl.BlockSpec((TB, D_PAD), lambda i: (i, 0)),
      out_shape=jax.ShapeDtypeStruct((VOCAB, D_PAD), jnp.float32),
  )(tt)
  return padded.reshape(2 * VOCAB, D_EMB)


BLK = 256


def _mlp_body(x, W1, b1, g1, be1, Wo, bo, o):
  h = jnp.dot(x[:, :], W1[:, :], preferred_element_type=jnp.float32)
  h = jnp.maximum(h + b1[:, :], 0.0)
  mu = jnp.mean(h, axis=-1, keepdims=True)
  var = jnp.mean((h - mu) ** 2, axis=-1, keepdims=True)
  hn = (h - mu) * lax.rsqrt(var + EPS) * g1[:, :] + be1[:, :]
  o[:, :] = jnp.dot(hn, Wo[:, :], preferred_element_type=jnp.float32) + bo[:, :]


def _mlp(x, W1, b1, g1, be1, Wo, bo):
  full = lambda shape: pl.BlockSpec(shape, lambda i: (0, 0))
  return pl.pallas_call(
      _mlp_body,
      grid=(B // BLK,),
      in_specs=[
          pl.BlockSpec((BLK, D_EMB), lambda i: (i, 0)),
          full((D_EMB, H)), full((1, H)), full((1, H)), full((1, H)),
          full((H, OUT)), full((1, OUT)),
      ],
      out_specs=pl.BlockSpec((BLK, OUT), lambda i: (i, 0)),
      out_shape=jax.ShapeDtypeStruct((B, OUT), jnp.float32),
  )(x, W1, b1, g1, be1, Wo, bo)


def kernel(query_tokens, doc_tokens, emb_table,
           W1_q, b1_q, g1_q, be1_q, Wo_q, bo_q,
           W1_d, b1_d, g1_d, be1_d, Wo_d, bo_d):
  qt = query_tokens.astype(jnp.int32) * 2
  dt = doc_tokens.astype(jnp.int32) * 2
  tbl = _repack(emb_table.T)
  pooled_d = _pool_d(dt, tbl)
  pooled_q = _pool_q(qt, tbl)
  row = lambda v: v.reshape(1, -1)
  d = _mlp(pooled_d, W1_d, row(b1_d), row(g1_d), row(be1_d), Wo_d, row(bo_d))
  q = _mlp(pooled_q, W1_q, row(b1_q), row(g1_q), row(be1_q), Wo_q, row(bo_q))
  return (q, d)
